# baseline (speedup 1.0000x reference)
import functools

import jax
import jax.numpy as jnp
from jax import lax
from jax.experimental import pallas as pl
from jax.experimental.pallas import tpu as pltpu
from jax.experimental.pallas import tpu_sc as plsc

N = 50000
NINP = 64
D1 = 128
D2 = 64
B, L = 1024, 50

NC, NS = 2, 16
NPAD = 50176
ROWS_PER_TILE = NPAD // NS

K = 1024
SUB = K // 128
NBLK = 50
EDGES_PER_TILE = NBLK * K
EPAD = EDGES_PER_TILE * NS
E = 800000
PADN = EPAD - E

CW = 16

GPT = 1600
BL = B * L

_mesh = plsc.VectorSubcoreMesh(core_axis_name="c", subcore_axis_name="s")
_sc_params = pltpu.CompilerParams(use_tc_tiling_on_sc=False)


_DROWS = EDGES_PER_TILE // 128


@functools.partial(
    pl.kernel,
    out_type=jax.ShapeDtypeStruct((NPAD, 8), jnp.float32),
    mesh=_mesh,
    compiler_params=_sc_params,
    scratch_types=[
        pltpu.VMEM((_DROWS, 128), jnp.int32),
        pltpu.VMEM((128, 8), jnp.float32),
        pltpu.VMEM_SHARED((NPAD, 8), jnp.float32),
        pltpu.SemaphoreType.DMA,
    ],
)
def _degree(dst2_hbm, ones_hbm, zeros8_hbm, deg_out, dstb, onesb, acc, sem):
  c = lax.axis_index("c")
  t = lax.axis_index("s")
  rbase = t * ROWS_PER_TILE

  @pl.when(c == 0)
  def _():
    pltpu.sync_copy(zeros8_hbm, acc.at[pl.ds(rbase, ROWS_PER_TILE)])
    pltpu.sync_copy(ones_hbm, onesb)
    pltpu.sync_copy(dst2_hbm.at[pl.ds(t * _DROWS, _DROWS)], dstb)
    plsc.subcore_barrier()

    lag = 8

    def blk(jj, carry):
      pltpu.async_copy(onesb, acc.at[dstb.at[jj]], sem, add=True)

      @pl.when(jj >= lag)
      def _():
        pltpu.make_async_copy(onesb, acc.at[dstb.at[0]], sem).wait()

      return carry

    lax.fori_loop(0, _DROWS, blk, 0)

    def tail(jj, carry):
      pltpu.make_async_copy(onesb, acc.at[dstb.at[0]], sem).wait()
      return carry

    lax.fori_loop(0, lag, tail, 0)
    plsc.subcore_barrier()
    pltpu.sync_copy(acc.at[pl.ds(rbase, ROWS_PER_TILE)],
                    deg_out.at[pl.ds(rbase, ROWS_PER_TILE)])


def _make_scatter(n_chunks):
  per_sc = n_chunks // NC

  @functools.partial(
      pl.kernel,
      out_type=jax.ShapeDtypeStruct((NPAD, n_chunks * CW), jnp.float32),
      mesh=_mesh,
      compiler_params=_sc_params,
      scratch_types=[
          pltpu.VMEM((K,), jnp.int32),
          pltpu.VMEM((K,), jnp.int32),
          pltpu.VMEM((SUB, 128), jnp.int32),
          pltpu.VMEM((SUB, 128), jnp.int32),
          pltpu.VMEM((K, CW), jnp.float32),
          pltpu.VMEM((K, CW), jnp.float32),
          pltpu.VMEM_SHARED((NPAD, CW), jnp.float32),
          pltpu.SemaphoreType.DMA,
          pltpu.SemaphoreType.DMA,
      ],
  )
  def scat(idx_hbm, dst2_hbm, yflat_hbm, zeros_hbm, s_out,
           idxb0, idxb1, dstb0, dstb1, rowb0, rowb1, acc,
           sem0, sem1):
    idxb = (idxb0, idxb1)
    dstb = (dstb0, dstb1)
    rowb = (rowb0, rowb1)
    sems = (sem0, sem1)
    c = lax.axis_index("c")
    t = lax.axis_index("s")
    rbase = t * ROWS_PER_TILE

    for cc in range(NC):
      @pl.when(c == cc)
      def _(cc=cc):
        for step in range(per_sc):
          pp = cc * per_sc + step
          pltpu.sync_copy(zeros_hbm, acc.at[pl.ds(rbase, ROWS_PER_TILE)])
          plsc.subcore_barrier()

          def load(k, b, pp=pp):
            off = t * EDGES_PER_TILE + k * K
            row0 = t * (EDGES_PER_TILE // 128) + k * SUB
            pltpu.sync_copy(idx_hbm.at[pp].at[pl.ds(off, K)], idxb[b])
            pltpu.sync_copy(dst2_hbm.at[pl.ds(row0, SUB)], dstb[b])

          def fire(b):
            for j in range(SUB):
              pltpu.async_copy(
                  yflat_hbm.at[idxb[b].at[pl.ds(j * 128, 128)]],
                  rowb[b].at[pl.ds(j * 128, 128)], sems[b])

          def drain(b):
            pltpu.make_async_copy(yflat_hbm.at[idxb[b]], rowb[b],
                                  sems[b]).wait()

          def scatter(b):
            for j in range(SUB):
              pltpu.sync_copy(rowb[b].at[pl.ds(j * 128, 128)],
                              acc.at[dstb[b].at[j]], add=True)

          load(0, 0)
          fire(0)

          def body2(i, carry):
            k0 = 2 * i
            load(k0 + 1, 1)
            fire(1)
            drain(0)
            scatter(0)

            @pl.when(i < NBLK // 2 - 1)
            def _():
              load(k0 + 2, 0)
              fire(0)

            drain(1)
            scatter(1)
            return carry

          lax.fori_loop(0, NBLK // 2, body2, 0)
          plsc.subcore_barrier()
          pltpu.sync_copy(acc.at[pl.ds(rbase, ROWS_PER_TILE)],
                          s_out.at[pl.ds(rbase, ROWS_PER_TILE),
                                   pl.ds(pp * CW, CW)])
          plsc.subcore_barrier()

  return scat


_scatter1 = _make_scatter(D1 // CW)
_scatter2 = _make_scatter(D2 // CW)


@functools.partial(
    pl.kernel,
    out_type=jax.ShapeDtypeStruct((BL, D2), jnp.float32),
    mesh=_mesh,
    compiler_params=_sc_params,
    scratch_types=[
        pltpu.VMEM((GPT,), jnp.int32),
        pltpu.VMEM((GPT, D2), jnp.float32),
        pltpu.SemaphoreType.DMA,
    ],
)
def _gather_rows(idx_hbm, table_hbm, out_hbm, idxb, rowb, sem):
  c = lax.axis_index("c")
  t = lax.axis_index("s")
  base = (c * NS + t) * GPT
  pltpu.sync_copy(idx_hbm.at[pl.ds(base, GPT)], idxb)
  for j in range(12):
    pltpu.async_copy(table_hbm.at[idxb.at[pl.ds(j * 128, 128)]],
                     rowb.at[pl.ds(j * 128, 128)], sem)
  pltpu.async_copy(table_hbm.at[idxb.at[pl.ds(1536, 64)]],
                   rowb.at[pl.ds(1536, 64)], sem)
  pltpu.make_async_copy(table_hbm.at[idxb], rowb, sem).wait()
  pltpu.sync_copy(rowb, out_hbm.at[pl.ds(base, GPT)])


_R = 2000
_EB = 8192


def _eidx_body(src_ref, i8_ref, i4_ref):
  s = src_ref[...]
  p = lax.broadcasted_iota(jnp.int32, (8, _EB), 0)
  i8_ref[...] = s[None, :] * 8 + p
  i4_ref[...] = s[None, :] * 4 + p % 4


def _d1_body(emb_ref, w1_ref, deg_ref, y_ref, dinv_ref):
  deg = deg_ref[:, 0:1] + 1.0
  dinv = lax.rsqrt(deg)
  y = jnp.dot(emb_ref[...], w1_ref[...], preferred_element_type=jnp.float32,
              precision=lax.Precision.HIGHEST)
  y_ref[...] = y * dinv
  dinv_ref[...] = jnp.broadcast_to(dinv, (_R, 8))


def _d2_body(s_ref, y_ref, dinv_ref, w2_ref, b1_ref, y2_ref):
  dinv = dinv_ref[:, 0:1]
  out1 = dinv * (s_ref[...] + y_ref[...]) + b1_ref[...]
  y2_ref[...] = dinv * jnp.dot(out1, w2_ref[...],
                               preferred_element_type=jnp.float32,
                               precision=lax.Precision.HIGHEST)


def _d3_body(s_ref, y_ref, dinv_ref, b2_ref, out_ref):
  dinv = dinv_ref[:, 0:1]
  out_ref[...] = dinv * (s_ref[...] + y_ref[...]) + b2_ref[...]


def _rows(i):
  return (i, 0)


def _const(i):
  return (0, 0)


def _edge_idx(src_p):
  return pl.pallas_call(
      _eidx_body,
      grid=(EPAD // _EB,),
      in_specs=[pl.BlockSpec((_EB,), lambda i: (i,))],
      out_specs=[pl.BlockSpec((8, _EB), lambda i: (0, i))] * 2,
      out_shape=[jax.ShapeDtypeStruct((8, EPAD), jnp.int32)] * 2,
  )(src_p)


def _dense1(emb, w1, deg):
  return pl.pallas_call(
      _d1_body,
      grid=(N // _R,),
      in_specs=[
          pl.BlockSpec((_R, NINP), _rows),
          pl.BlockSpec((NINP, D1), _const),
          pl.BlockSpec((_R, 8), _rows),
      ],
      out_specs=[
          pl.BlockSpec((_R, D1), _rows),
          pl.BlockSpec((_R, 8), _rows),
      ],
      out_shape=[
          jax.ShapeDtypeStruct((N, D1), jnp.float32),
          jax.ShapeDtypeStruct((N, 8), jnp.float32),
      ],
  )(emb, w1, deg)


def _dense2(s1, y1, dinv, w2, b1):
  return pl.pallas_call(
      _d2_body,
      grid=(N // _R,),
      in_specs=[
          pl.BlockSpec((_R, D1), _rows),
          pl.BlockSpec((_R, D1), _rows),
          pl.BlockSpec((_R, 8), _rows),
          pl.BlockSpec((D1, D2), _const),
          pl.BlockSpec((1, D1), _const),
      ],
      out_specs=pl.BlockSpec((_R, D2), _rows),
      out_shape=jax.ShapeDtypeStruct((N, D2), jnp.float32),
  )(s1, y1, dinv, w2, b1)


def _dense3(s2, y2, dinv, b2):
  return pl.pallas_call(
      _d3_body,
      grid=(N // _R,),
      in_specs=[
          pl.BlockSpec((_R, D2), _rows),
          pl.BlockSpec((_R, D2), _rows),
          pl.BlockSpec((_R, 8), _rows),
          pl.BlockSpec((1, D2), _const),
      ],
      out_specs=pl.BlockSpec((_R, D2), _rows),
      out_shape=jax.ShapeDtypeStruct((N, D2), jnp.float32),
  )(s2, y2, dinv, b2)


def kernel(input, input_timestamp, edge_index, emb, W1, b1, W2, b2):
  del input_timestamp
  src = edge_index[0].astype(jnp.int32)
  dst = edge_index[1].astype(jnp.int32)
  pi = lax.iota(jnp.int32, PADN)
  src_p = jnp.concatenate([src, (pi * 37) % N])
  dst_p = jnp.concatenate([dst, N + pi % (NPAD - N)])
  dst2 = dst_p.reshape(EPAD // 128, 128)

  ones8 = jnp.ones((128, 8), jnp.float32)
  zeros8 = jnp.zeros((ROWS_PER_TILE, 8), jnp.float32)
  zerosC = jnp.zeros((ROWS_PER_TILE, CW), jnp.float32)

  idx8, idx4 = _edge_idx(src_p)
  deg = _degree(dst2, ones8, zeros8)
  y1, dinv = _dense1(emb, W1, deg)
  s1 = _scatter1(idx8, dst2, y1.reshape(N * (D1 // CW), CW), zerosC)
  y2 = _dense2(s1, y1, dinv, W2, b1.reshape(1, D1))
  s2 = _scatter2(idx4, dst2, y2.reshape(N * (D2 // CW), CW), zerosC)
  out2 = _dense3(s2, y2, dinv, b2.reshape(1, D2))

  rows = _gather_rows(input.reshape(BL).astype(jnp.int32), out2)
  return rows.reshape(B, L, D2)


# speedup vs baseline: 1.0974x; 1.0974x over previous
import functools

import jax
import jax.numpy as jnp
from jax import lax
from jax.experimental import pallas as pl
from jax.experimental.pallas import tpu as pltpu
from jax.experimental.pallas import tpu_sc as plsc

N = 50000
NINP = 64
D1 = 128
D2 = 64
B, L = 1024, 50

NC, NS = 2, 16
NPAD = 50176
ROWS_PER_TILE = NPAD // NS

K = 2048
SUB = K // 128
NBLK = 26
EDGES_PER_TILE = NBLK * K
EPAD = EDGES_PER_TILE * NS
E = 800000
PADN = EPAD - E

CW = 16

GPT = 1600
BL = B * L

_mesh = plsc.VectorSubcoreMesh(core_axis_name="c", subcore_axis_name="s")
_sc_params = pltpu.CompilerParams(use_tc_tiling_on_sc=False)


_DROWS = EDGES_PER_TILE // 128


@functools.partial(
    pl.kernel,
    out_type=jax.ShapeDtypeStruct((NPAD, 8), jnp.float32),
    mesh=_mesh,
    compiler_params=_sc_params,
    scratch_types=[
        pltpu.VMEM((_DROWS, 128), jnp.int32),
        pltpu.VMEM((128, 8), jnp.float32),
        pltpu.VMEM_SHARED((NPAD, 8), jnp.float32),
        pltpu.SemaphoreType.DMA,
    ],
)
def _degree(dst2_hbm, ones_hbm, zeros8_hbm, deg_out, dstb, onesb, acc, sem):
  c = lax.axis_index("c")
  t = lax.axis_index("s")
  rbase = t * ROWS_PER_TILE

  @pl.when(c == 0)
  def _():
    pltpu.sync_copy(zeros8_hbm, acc.at[pl.ds(rbase, ROWS_PER_TILE)])
    pltpu.sync_copy(ones_hbm, onesb)
    pltpu.sync_copy(dst2_hbm.at[pl.ds(t * _DROWS, _DROWS)], dstb)
    plsc.subcore_barrier()

    lag = 8

    def blk(jj, carry):
      pltpu.async_copy(onesb, acc.at[dstb.at[jj]], sem, add=True)

      @pl.when(jj >= lag)
      def _():
        pltpu.make_async_copy(onesb, acc.at[dstb.at[0]], sem).wait()

      return carry

    lax.fori_loop(0, _DROWS, blk, 0)

    def tail(jj, carry):
      pltpu.make_async_copy(onesb, acc.at[dstb.at[0]], sem).wait()
      return carry

    lax.fori_loop(0, lag, tail, 0)
    plsc.subcore_barrier()
    pltpu.sync_copy(acc.at[pl.ds(rbase, ROWS_PER_TILE)],
                    deg_out.at[pl.ds(rbase, ROWS_PER_TILE)])


def _make_scatter(n_chunks):
  per_sc = n_chunks // NC

  @functools.partial(
      pl.kernel,
      out_type=jax.ShapeDtypeStruct((NPAD, n_chunks * CW), jnp.float32),
      mesh=_mesh,
      compiler_params=_sc_params,
      scratch_types=[
          pltpu.VMEM((K,), jnp.int32),
          pltpu.VMEM((K,), jnp.int32),
          pltpu.VMEM((SUB, 128), jnp.int32),
          pltpu.VMEM((SUB, 128), jnp.int32),
          pltpu.VMEM((K, CW), jnp.float32),
          pltpu.VMEM((K, CW), jnp.float32),
          pltpu.VMEM_SHARED((NPAD, CW), jnp.float32),
          pltpu.SemaphoreType.DMA,
          pltpu.SemaphoreType.DMA,
      ],
  )
  def scat(idx_hbm, dst2_hbm, yflat_hbm, zeros_hbm, s_out,
           idxb0, idxb1, dstb0, dstb1, rowb0, rowb1, acc,
           sem0, sem1):
    idxb = (idxb0, idxb1)
    dstb = (dstb0, dstb1)
    rowb = (rowb0, rowb1)
    sems = (sem0, sem1)
    c = lax.axis_index("c")
    t = lax.axis_index("s")
    rbase = t * ROWS_PER_TILE

    for cc in range(NC):
      @pl.when(c == cc)
      def _(cc=cc):
        for step in range(per_sc):
          pp = cc * per_sc + step
          pltpu.sync_copy(zeros_hbm, acc.at[pl.ds(rbase, ROWS_PER_TILE)])
          plsc.subcore_barrier()

          def load(k, b, pp=pp):
            off = t * EDGES_PER_TILE + k * K
            row0 = t * (EDGES_PER_TILE // 128) + k * SUB
            pltpu.sync_copy(idx_hbm.at[pp].at[pl.ds(off, K)], idxb[b])
            pltpu.sync_copy(dst2_hbm.at[pl.ds(row0, SUB)], dstb[b])

          def fire(b):
            for j in range(SUB):
              pltpu.async_copy(
                  yflat_hbm.at[idxb[b].at[pl.ds(j * 128, 128)]],
                  rowb[b].at[pl.ds(j * 128, 128)], sems[b])

          def drain(b):
            pltpu.make_async_copy(yflat_hbm.at[idxb[b]], rowb[b],
                                  sems[b]).wait()

          def scatter(b):
            for j in range(SUB):
              pltpu.sync_copy(rowb[b].at[pl.ds(j * 128, 128)],
                              acc.at[dstb[b].at[j]], add=True)

          load(0, 0)
          fire(0)

          def body2(i, carry):
            k0 = 2 * i
            load(k0 + 1, 1)
            fire(1)
            drain(0)
            scatter(0)

            @pl.when(i < NBLK // 2 - 1)
            def _():
              load(k0 + 2, 0)
              fire(0)

            drain(1)
            scatter(1)
            return carry

          lax.fori_loop(0, NBLK // 2, body2, 0)
          plsc.subcore_barrier()
          pltpu.sync_copy(acc.at[pl.ds(rbase, ROWS_PER_TILE)],
                          s_out.at[pl.ds(rbase, ROWS_PER_TILE),
                                   pl.ds(pp * CW, CW)])
          plsc.subcore_barrier()

  return scat


_scatter1 = _make_scatter(D1 // CW)
_scatter2 = _make_scatter(D2 // CW)


@functools.partial(
    pl.kernel,
    out_type=jax.ShapeDtypeStruct((BL, D2), jnp.float32),
    mesh=_mesh,
    compiler_params=_sc_params,
    scratch_types=[
        pltpu.VMEM((GPT,), jnp.int32),
        pltpu.VMEM((GPT, D2), jnp.float32),
        pltpu.SemaphoreType.DMA,
    ],
)
def _gather_rows(idx_hbm, table_hbm, out_hbm, idxb, rowb, sem):
  c = lax.axis_index("c")
  t = lax.axis_index("s")
  base = (c * NS + t) * GPT
  pltpu.sync_copy(idx_hbm.at[pl.ds(base, GPT)], idxb)
  for j in range(12):
    pltpu.async_copy(table_hbm.at[idxb.at[pl.ds(j * 128, 128)]],
                     rowb.at[pl.ds(j * 128, 128)], sem)
  pltpu.async_copy(table_hbm.at[idxb.at[pl.ds(1536, 64)]],
                   rowb.at[pl.ds(1536, 64)], sem)
  pltpu.make_async_copy(table_hbm.at[idxb], rowb, sem).wait()
  pltpu.sync_copy(rowb, out_hbm.at[pl.ds(base, GPT)])


_R = 2000
_EB = 8192


def _eidx_body(src_ref, i8_ref, i4_ref):
  s = src_ref[...]
  p = lax.broadcasted_iota(jnp.int32, (8, _EB), 0)
  i8_ref[...] = s[None, :] * 8 + p
  i4_ref[...] = s[None, :] * 4 + p % 4


def _d1_body(emb_ref, w1_ref, deg_ref, y_ref, dinv_ref):
  deg = deg_ref[:, 0:1] + 1.0
  dinv = lax.rsqrt(deg)
  y = jnp.dot(emb_ref[...], w1_ref[...], preferred_element_type=jnp.float32,
              precision=lax.Precision.HIGHEST)
  y_ref[...] = y * dinv
  dinv_ref[...] = jnp.broadcast_to(dinv, (_R, 8))


def _d2_body(s_ref, y_ref, dinv_ref, w2_ref, b1_ref, y2_ref):
  dinv = dinv_ref[:, 0:1]
  out1 = dinv * (s_ref[...] + y_ref[...]) + b1_ref[...]
  y2_ref[...] = dinv * jnp.dot(out1, w2_ref[...],
                               preferred_element_type=jnp.float32,
                               precision=lax.Precision.HIGHEST)


def _d3_body(s_ref, y_ref, dinv_ref, b2_ref, out_ref):
  dinv = dinv_ref[:, 0:1]
  out_ref[...] = dinv * (s_ref[...] + y_ref[...]) + b2_ref[...]


def _rows(i):
  return (i, 0)


def _const(i):
  return (0, 0)


def _edge_idx(src_p):
  return pl.pallas_call(
      _eidx_body,
      grid=(EPAD // _EB,),
      in_specs=[pl.BlockSpec((_EB,), lambda i: (i,))],
      out_specs=[pl.BlockSpec((8, _EB), lambda i: (0, i))] * 2,
      out_shape=[jax.ShapeDtypeStruct((8, EPAD), jnp.int32)] * 2,
  )(src_p)


def _dense1(emb, w1, deg):
  return pl.pallas_call(
      _d1_body,
      grid=(N // _R,),
      in_specs=[
          pl.BlockSpec((_R, NINP), _rows),
          pl.BlockSpec((NINP, D1), _const),
          pl.BlockSpec((_R, 8), _rows),
      ],
      out_specs=[
          pl.BlockSpec((_R, D1), _rows),
          pl.BlockSpec((_R, 8), _rows),
      ],
      out_shape=[
          jax.ShapeDtypeStruct((N, D1), jnp.float32),
          jax.ShapeDtypeStruct((N, 8), jnp.float32),
      ],
  )(emb, w1, deg)


def _dense2(s1, y1, dinv, w2, b1):
  return pl.pallas_call(
      _d2_body,
      grid=(N // _R,),
      in_specs=[
          pl.BlockSpec((_R, D1), _rows),
          pl.BlockSpec((_R, D1), _rows),
          pl.BlockSpec((_R, 8), _rows),
          pl.BlockSpec((D1, D2), _const),
          pl.BlockSpec((1, D1), _const),
      ],
      out_specs=pl.BlockSpec((_R, D2), _rows),
      out_shape=jax.ShapeDtypeStruct((N, D2), jnp.float32),
  )(s1, y1, dinv, w2, b1)


def _dense3(s2, y2, dinv, b2):
  return pl.pallas_call(
      _d3_body,
      grid=(N // _R,),
      in_specs=[
          pl.BlockSpec((_R, D2), _rows),
          pl.BlockSpec((_R, D2), _rows),
          pl.BlockSpec((_R, 8), _rows),
          pl.BlockSpec((1, D2), _const),
      ],
      out_specs=pl.BlockSpec((_R, D2), _rows),
      out_shape=jax.ShapeDtypeStruct((N, D2), jnp.float32),
  )(s2, y2, dinv, b2)


def kernel(input, input_timestamp, edge_index, emb, W1, b1, W2, b2):
  del input_timestamp
  src = edge_index[0].astype(jnp.int32)
  dst = edge_index[1].astype(jnp.int32)
  pi = lax.iota(jnp.int32, PADN)
  src_p = jnp.concatenate([src, (pi * 37) % N])
  dst_p = jnp.concatenate([dst, N + pi % (NPAD - N)])
  dst2 = dst_p.reshape(EPAD // 128, 128)

  ones8 = jnp.ones((128, 8), jnp.float32)
  zeros8 = jnp.zeros((ROWS_PER_TILE, 8), jnp.float32)
  zerosC = jnp.zeros((ROWS_PER_TILE, CW), jnp.float32)

  idx8, idx4 = _edge_idx(src_p)
  deg = _degree(dst2, ones8, zeros8)
  y1, dinv = _dense1(emb, W1, deg)
  s1 = _scatter1(idx8, dst2, y1.reshape(N * (D1 // CW), CW), zerosC)
  y2 = _dense2(s1, y1, dinv, W2, b1.reshape(1, D1))
  s2 = _scatter2(idx4, dst2, y2.reshape(N * (D2 // CW), CW), zerosC)
  out2 = _dense3(s2, y2, dinv, b2.reshape(1, D2))

  rows = _gather_rows(input.reshape(BL).astype(jnp.int32), out2)
  return rows.reshape(B, L, D2)
